# baseline (speedup 1.0000x reference)
import functools

import jax
import jax.numpy as jnp
from jax import lax
from jax.experimental import pallas as pl
from jax.experimental.pallas import tpu as pltpu
from jax.experimental.pallas import tpu_sc as plsc

N = 10000
E = 320000
D = 128
N_LAYERS = 3
ALPHA = sum(1.0 / (1 + k) for k in range(N_LAYERS))

NC, NS, L = 2, 16, 16
NW = NC * NS
EPW = E // NW
C = 64
EPAD = 240
EPWP = EPW + EPAD
NCHUNK = EPWP // C
NACC = 10080
DB = 80
NZB = NACC // DB
NDB = N // DB

_mesh = plsc.VectorSubcoreMesh(
    core_axis_name="c", subcore_axis_name="s", num_cores=NC, num_subcores=NS
)
_sc_params = pltpu.CompilerParams(
    needs_layout_passes=False, use_tc_tiling_on_sc=False
)


@functools.partial(
    pl.kernel,
    out_type=jax.ShapeDtypeStruct((NW, 2, N), jnp.float32),
    mesh=_mesh,
    scratch_types=[
        pltpu.VMEM((EPW,), jnp.int32),
        pltpu.VMEM((N,), jnp.float32),
        pltpu.VMEM((N,), jnp.float32),
    ],
    compiler_params=_sc_params,
)
def _deg_kernel(src_hbm, dst_hbm, deg_hbm, ev, cnt_s, cnt_d):
    c = lax.axis_index("c")
    s = lax.axis_index("s")
    wid = c * NS + s
    base = wid * EPW
    zeros = jnp.zeros((L,), jnp.float32)
    ones = jnp.ones((L,), jnp.float32)

    def zero_body(g, _):
        cnt_s[pl.ds(g * L, L)] = zeros
        cnt_d[pl.ds(g * L, L)] = zeros
        return 0

    lax.fori_loop(0, N // L, zero_body, 0)

    for cnt, eh in ((cnt_s, src_hbm), (cnt_d, dst_hbm)):
        pltpu.sync_copy(eh.at[pl.ds(base, EPW)], ev)

        def acc_body(g, _, cnt=cnt):
            idx = ev[pl.ds(g * L, L)]
            plsc.addupdate_scatter(cnt, [idx], ones)
            return 0

        lax.fori_loop(0, EPW // L, acc_body, 0)

    pltpu.sync_copy(cnt_s, deg_hbm.at[wid, 0])
    pltpu.sync_copy(cnt_d, deg_hbm.at[wid, 1])


def _norm_body(emb_ref, deg_ref, m_ref, innorm_ref):
    out_deg = jnp.sum(deg_ref[:, 0, :], axis=0)
    in_deg = jnp.sum(deg_ref[:, 1, :], axis=0)
    out_norm = lax.rsqrt(jnp.maximum(out_deg, 1.0))
    innorm_ref[...] = lax.rsqrt(jnp.maximum(in_deg, 1.0))
    m_ref[...] = (emb_ref[...] * out_norm[:, None]).astype(jnp.bfloat16)


def _norm_call(emb, degs):
    return pl.pallas_call(
        _norm_body,
        out_shape=[
            jax.ShapeDtypeStruct((N, D), jnp.bfloat16),
            jax.ShapeDtypeStruct((N,), jnp.float32),
        ],
    )(emb, degs)


@functools.partial(
    pl.kernel,
    out_type=jax.ShapeDtypeStruct((NC, N, D), jnp.bfloat16),
    mesh=_mesh,
    scratch_types=[
        pltpu.VMEM_SHARED((NACC, D), jnp.bfloat16),
        pltpu.VMEM((NCHUNK, C), jnp.int32),
        pltpu.VMEM((NCHUNK, C), jnp.int32),
        pltpu.VMEM((C, D), jnp.bfloat16),
        pltpu.VMEM((C, D), jnp.bfloat16),
        pltpu.VMEM((C, D), jnp.bfloat16),
        pltpu.VMEM((DB, D), jnp.bfloat16),
        pltpu.SemaphoreType.DMA,
        pltpu.SemaphoreType.DMA,
        pltpu.SemaphoreType.DMA,
    ],
    compiler_params=_sc_params,
)
def _scatter_kernel(
    m_hbm, src_hbm, dst_hbm, part_hbm,
    acc, src_all, dst_all, rows0, rows1, rows2, zdbuf, sem0, sem1, sem2,
):
    c = lax.axis_index("c")
    s = lax.axis_index("s")
    wid = c * NS + s
    zeros = jnp.zeros((2 * L,), jnp.bfloat16)

    pltpu.sync_copy(src_hbm.at[pl.ds(wid * NCHUNK, NCHUNK)], src_all)
    pltpu.sync_copy(dst_hbm.at[pl.ds(wid * NCHUNK, NCHUNK)], dst_all)

    def zb_body(i, _):
        def zb_inner(j, _):
            zdbuf[i, pl.ds(j * 2 * L, 2 * L)] = zeros
            return 0

        lax.fori_loop(0, D // (2 * L), zb_inner, 0)
        return 0

    lax.fori_loop(0, DB, zb_body, 0)

    for q in range((NZB + NS - 1) // NS):
        t = s + NS * q

        @pl.when(t < NZB)
        def _():
            pltpu.sync_copy(zdbuf, acc.at[pl.ds(t * DB, DB)])

    plsc.subcore_barrier()

    bufs = (rows0, rows1, rows2)
    sems = (sem0, sem1, sem2)
    NB = 3

    def gstart(k, b):
        pltpu.async_copy(m_hbm.at[src_all.at[k]], bufs[b], sems[b])

    def gwait(b):
        pltpu.make_async_copy(m_hbm.at[src_all.at[0]], bufs[b], sems[b]).wait()

    def scat(k, b):
        pltpu.sync_copy(bufs[b], acc.at[dst_all.at[k]], add=True)

    gstart(0, 0)
    gstart(1, 1)

    def ring_body(q, _):
        for r in range(NB):
            k = NB * q + r
            gwait(r)

            @pl.when(k + NB - 1 < NCHUNK)
            def _():
                gstart(k + NB - 1, (r + NB - 1) % NB)

            scat(k, r)
        return 0

    lax.fori_loop(0, NCHUNK // NB, ring_body, 0)
    for k in range((NCHUNK // NB) * NB, NCHUNK):
        gwait(k % NB)
        scat(k, k % NB)
    plsc.subcore_barrier()

    for q in range((NDB + NS - 1) // NS):
        t = s + NS * q

        @pl.when(t < NDB)
        def _():
            pltpu.sync_copy(acc.at[pl.ds(t * DB, DB)], zdbuf)
            pltpu.sync_copy(zdbuf, part_hbm.at[c, pl.ds(t * DB, DB)])


def _final_body(part_ref, innorm_ref, out_ref):
    agg = part_ref[0].astype(jnp.float32) + part_ref[1].astype(jnp.float32)
    out_ref[...] = agg * (innorm_ref[...] * ALPHA)[:, None]


def _final_call(parts, in_norm):
    return pl.pallas_call(
        _final_body,
        out_shape=jax.ShapeDtypeStruct((N, D), jnp.float32),
    )(parts, in_norm)


def kernel(emb, edge_index):
    src = edge_index[0]
    dst = edge_index[1]
    degs = _deg_kernel(src, dst)
    m, in_norm = _norm_call(emb, degs)
    src2d = jnp.pad(src.reshape(NW, EPW), ((0, 0), (0, EPAD))).reshape(
        NW * NCHUNK, C
    )
    dst2d = jnp.pad(
        dst.reshape(NW, EPW), ((0, 0), (0, EPAD)), constant_values=N
    ).reshape(NW * NCHUNK, C)
    parts = _scatter_kernel(m, src2d, dst2d)
    return _final_call(parts, in_norm)


# speedup vs baseline: 1.9265x; 1.9265x over previous
import functools

import jax
import jax.numpy as jnp
from jax import lax
from jax.experimental import pallas as pl
from jax.experimental.pallas import tpu as pltpu
from jax.experimental.pallas import tpu_sc as plsc

N = 10000
E = 320000
D = 128
N_LAYERS = 3
ALPHA = sum(1.0 / (1 + k) for k in range(N_LAYERS))

NC, NS, L = 2, 16, 16
NW = NC * NS
EPW = E // NW
C = 80
NCHUNK = EPW // C
RPT = N // NS
ZR = 125
NZ = RPT // ZR

_mesh = plsc.VectorSubcoreMesh(
    core_axis_name="c", subcore_axis_name="s", num_cores=NC, num_subcores=NS
)
_sc_params = pltpu.CompilerParams(
    needs_layout_passes=False, use_tc_tiling_on_sc=False
)


@functools.partial(
    pl.kernel,
    out_type=jax.ShapeDtypeStruct((NW, 2, N), jnp.float32),
    mesh=_mesh,
    scratch_types=[
        pltpu.VMEM((EPW,), jnp.int32),
        pltpu.VMEM((N,), jnp.float32),
        pltpu.VMEM((N,), jnp.float32),
    ],
    compiler_params=_sc_params,
)
def _deg_kernel(src_hbm, dst_hbm, deg_hbm, ev, cnt_s, cnt_d):
    c = lax.axis_index("c")
    s = lax.axis_index("s")
    wid = c * NS + s
    base = wid * EPW
    zeros = jnp.zeros((L,), jnp.float32)
    ones = jnp.ones((L,), jnp.float32)

    def zero_body(g, _):
        cnt_s[pl.ds(g * L, L)] = zeros
        cnt_d[pl.ds(g * L, L)] = zeros
        return 0

    lax.fori_loop(0, N // L, zero_body, 0)

    for cnt, eh in ((cnt_s, src_hbm), (cnt_d, dst_hbm)):
        pltpu.sync_copy(eh.at[pl.ds(base, EPW)], ev)

        def acc_body(g, _, cnt=cnt):
            idx = ev[pl.ds(g * L, L)]
            plsc.addupdate_scatter(cnt, [idx], ones)
            return 0

        lax.fori_loop(0, EPW // L, acc_body, 0)

    pltpu.sync_copy(cnt_s, deg_hbm.at[wid, 0])
    pltpu.sync_copy(cnt_d, deg_hbm.at[wid, 1])


def _norm_body(emb_ref, deg_ref, m_ref, innorm_ref):
    out_deg = jnp.sum(deg_ref[:, 0, :], axis=0)
    in_deg = jnp.sum(deg_ref[:, 1, :], axis=0)
    out_norm = lax.rsqrt(jnp.maximum(out_deg, 1.0))
    innorm_ref[...] = lax.rsqrt(jnp.maximum(in_deg, 1.0))
    m_ref[...] = emb_ref[...] * out_norm[:, None]


def _norm_call(emb, degs):
    return pl.pallas_call(
        _norm_body,
        out_shape=[
            jax.ShapeDtypeStruct((N, D), jnp.float32),
            jax.ShapeDtypeStruct((N,), jnp.float32),
        ],
    )(emb, degs)


@functools.partial(
    pl.kernel,
    out_type=jax.ShapeDtypeStruct((NC, N, D), jnp.float32),
    mesh=_mesh,
    scratch_types=[
        pltpu.VMEM_SHARED((N, D), jnp.float32),
        pltpu.VMEM((NCHUNK, C), jnp.int32),
        pltpu.VMEM((NCHUNK, C), jnp.int32),
        pltpu.VMEM((C, D), jnp.float32),
        pltpu.VMEM((C, D), jnp.float32),
        pltpu.VMEM((C, D), jnp.float32),
        pltpu.SemaphoreType.DMA,
        pltpu.SemaphoreType.DMA,
        pltpu.SemaphoreType.DMA,
    ],
    compiler_params=_sc_params,
)
def _scatter_kernel(
    m_hbm, src_hbm, dst_hbm, part_hbm,
    acc, src_all, dst_all, rows0, rows1, rows2, sem0, sem1, sem2,
):
    c = lax.axis_index("c")
    s = lax.axis_index("s")
    wid = c * NS + s
    zeros = jnp.zeros((L,), jnp.float32)

    pltpu.sync_copy(src_hbm.at[pl.ds(wid * NCHUNK, NCHUNK)], src_all)
    pltpu.sync_copy(dst_hbm.at[pl.ds(wid * NCHUNK, NCHUNK)], dst_all)

    def zb_body(i, _):
        def zb_inner(j, _):
            rows0[i, pl.ds(j * L, L)] = zeros
            return 0

        lax.fori_loop(0, D // L, zb_inner, 0)
        return 0

    lax.fori_loop(0, C, zb_body, 0)

    for q in range((NCHUNK + NS - 1) // NS):
        t = s + NS * q

        @pl.when(t < NCHUNK)
        def _():
            pltpu.sync_copy(rows0, acc.at[pl.ds(t * C, C)])

    plsc.subcore_barrier()

    bufs = (rows0, rows1, rows2)
    sems = (sem0, sem1, sem2)
    NB = 3

    def gstart(k, b):
        pltpu.async_copy(m_hbm.at[src_all.at[k]], bufs[b], sems[b])

    def gwait(b):
        pltpu.make_async_copy(m_hbm.at[src_all.at[0]], bufs[b], sems[b]).wait()

    def scat(k, b):
        pltpu.sync_copy(bufs[b], acc.at[dst_all.at[k]], add=True)

    gstart(0, 0)
    gstart(1, 1)

    def ring_body(q, _):
        for r in range(NB):
            k = NB * q + r
            gwait(r)

            @pl.when(k + NB - 1 < NCHUNK)
            def _():
                gstart(k + NB - 1, (r + NB - 1) % NB)

            scat(k, r)
        return 0

    lax.fori_loop(0, NCHUNK // NB, ring_body, 0)
    for k in range((NCHUNK // NB) * NB, NCHUNK):
        gwait(k % NB)
        scat(k, k % NB)
    plsc.subcore_barrier()

    for q in range((NCHUNK + NS - 1) // NS):
        t = s + NS * q

        @pl.when(t < NCHUNK)
        def _():
            pltpu.sync_copy(acc.at[pl.ds(t * C, C)], rows0)
            pltpu.sync_copy(rows0, part_hbm.at[c, pl.ds(t * C, C)])


def _final_body(part_ref, innorm_ref, out_ref):
    agg = part_ref[0] + part_ref[1]
    out_ref[...] = agg * (innorm_ref[...] * ALPHA)[:, None]


def _final_call(parts, in_norm):
    return pl.pallas_call(
        _final_body,
        out_shape=jax.ShapeDtypeStruct((N, D), jnp.float32),
    )(parts, in_norm)


def kernel(emb, edge_index):
    src = edge_index[0]
    dst = edge_index[1]
    degs = _deg_kernel(src, dst)
    m, in_norm = _norm_call(emb, degs)
    src2d = src.reshape(E // C, C)
    dst2d = dst.reshape(E // C, C)
    parts = _scatter_kernel(m, src2d, dst2d)
    return _final_call(parts, in_norm)


# speedup vs baseline: 1.9838x; 1.0297x over previous
import functools

import jax
import jax.numpy as jnp
from jax import lax
from jax.experimental import pallas as pl
from jax.experimental.pallas import tpu as pltpu
from jax.experimental.pallas import tpu_sc as plsc

N = 10000
E = 320000
D = 128
N_LAYERS = 3
ALPHA = sum(1.0 / (1 + k) for k in range(N_LAYERS))

NC, NS, L = 2, 16, 16
NW = NC * NS
EPW = E // NW
C = 80
NCHUNK = EPW // C
RPT = N // NS
ZR = 125
NZ = RPT // ZR

_mesh = plsc.VectorSubcoreMesh(
    core_axis_name="c", subcore_axis_name="s", num_cores=NC, num_subcores=NS
)
_sc_params = pltpu.CompilerParams(
    needs_layout_passes=False, use_tc_tiling_on_sc=False
)


@functools.partial(
    pl.kernel,
    out_type=jax.ShapeDtypeStruct((NW, 2, N), jnp.float32),
    mesh=_mesh,
    scratch_types=[
        pltpu.VMEM((EPW,), jnp.int32),
        pltpu.VMEM((N,), jnp.float32),
        pltpu.VMEM((N,), jnp.float32),
    ],
    compiler_params=_sc_params,
)
def _deg_kernel(src_hbm, dst_hbm, deg_hbm, ev, cnt_s, cnt_d):
    c = lax.axis_index("c")
    s = lax.axis_index("s")
    wid = c * NS + s
    base = wid * EPW
    zeros = jnp.zeros((L,), jnp.float32)
    ones = jnp.ones((L,), jnp.float32)

    def zero_body(g, _):
        for u in range(5):
            cnt_s[pl.ds((5 * g + u) * L, L)] = zeros
            cnt_d[pl.ds((5 * g + u) * L, L)] = zeros
        return 0

    lax.fori_loop(0, N // L // 5, zero_body, 0)

    for cnt, eh in ((cnt_s, src_hbm), (cnt_d, dst_hbm)):
        pltpu.sync_copy(eh.at[pl.ds(base, EPW)], ev)

        def acc_body(g, _, cnt=cnt):
            for u in range(5):
                idx = ev[pl.ds((5 * g + u) * L, L)]
                plsc.addupdate_scatter(cnt, [idx], ones)
            return 0

        lax.fori_loop(0, EPW // L // 5, acc_body, 0)

    pltpu.sync_copy(cnt_s, deg_hbm.at[wid, 0])
    pltpu.sync_copy(cnt_d, deg_hbm.at[wid, 1])


def _norm_body(emb_ref, deg_ref, m_ref, innorm_ref):
    out_deg = jnp.sum(deg_ref[:, 0, :], axis=0)
    in_deg = jnp.sum(deg_ref[:, 1, :], axis=0)
    out_norm = lax.rsqrt(jnp.maximum(out_deg, 1.0))
    innorm_ref[...] = lax.rsqrt(jnp.maximum(in_deg, 1.0))
    m_ref[...] = emb_ref[...] * out_norm[:, None]


def _norm_call(emb, degs):
    return pl.pallas_call(
        _norm_body,
        out_shape=[
            jax.ShapeDtypeStruct((N, D), jnp.float32),
            jax.ShapeDtypeStruct((N,), jnp.float32),
        ],
    )(emb, degs)


@functools.partial(
    pl.kernel,
    out_type=jax.ShapeDtypeStruct((NC, N, D), jnp.float32),
    mesh=_mesh,
    scratch_types=[
        pltpu.VMEM_SHARED((N, D), jnp.float32),
        pltpu.VMEM((NCHUNK, C), jnp.int32),
        pltpu.VMEM((NCHUNK, C), jnp.int32),
        pltpu.VMEM((C, D), jnp.float32),
        pltpu.VMEM((C, D), jnp.float32),
        pltpu.VMEM((C, D), jnp.float32),
        pltpu.SemaphoreType.DMA,
        pltpu.SemaphoreType.DMA,
        pltpu.SemaphoreType.DMA,
    ],
    compiler_params=_sc_params,
)
def _scatter_kernel(
    m_hbm, src_hbm, dst_hbm, part_hbm,
    acc, src_all, dst_all, rows0, rows1, rows2, sem0, sem1, sem2,
):
    c = lax.axis_index("c")
    s = lax.axis_index("s")
    wid = c * NS + s
    zeros = jnp.zeros((L,), jnp.float32)

    pltpu.sync_copy(src_hbm.at[pl.ds(wid * NCHUNK, NCHUNK)], src_all)
    pltpu.sync_copy(dst_hbm.at[pl.ds(wid * NCHUNK, NCHUNK)], dst_all)

    def zb_body(i, _):
        def zb_inner(j, _):
            rows0[i, pl.ds(j * L, L)] = zeros
            return 0

        lax.fori_loop(0, D // L, zb_inner, 0)
        return 0

    lax.fori_loop(0, C, zb_body, 0)

    for q in range((NCHUNK + NS - 1) // NS):
        t = s + NS * q

        @pl.when(t < NCHUNK)
        def _():
            pltpu.async_copy(rows0, acc.at[pl.ds(t * C, C)], sem2)

    for q in range((NCHUNK + NS - 1) // NS):
        t = s + NS * q

        @pl.when(t < NCHUNK)
        def _():
            pltpu.make_async_copy(rows0, acc.at[pl.ds(0, C)], sem2).wait()

    plsc.subcore_barrier()

    bufs = (rows0, rows1, rows2)
    sems = (sem0, sem1, sem2)
    NB = 3

    def gstart(k, b):
        pltpu.async_copy(m_hbm.at[src_all.at[k]], bufs[b], sems[b])

    def gwait(b):
        pltpu.make_async_copy(m_hbm.at[src_all.at[0]], bufs[b], sems[b]).wait()

    def scat(k, b):
        pltpu.sync_copy(bufs[b], acc.at[dst_all.at[k]], add=True)

    gstart(0, 0)
    gstart(1, 1)

    def ring_body(q, _):
        for r in range(NB):
            k = NB * q + r
            gwait(r)

            @pl.when(k + NB - 1 < NCHUNK)
            def _():
                gstart(k + NB - 1, (r + NB - 1) % NB)

            scat(k, r)
        return 0

    lax.fori_loop(0, NCHUNK // NB, ring_body, 0)
    for k in range((NCHUNK // NB) * NB, NCHUNK):
        gwait(k % NB)
        scat(k, k % NB)
    plsc.subcore_barrier()

    NQ = (NCHUNK + NS - 1) // NS
    for q in range(NQ):
        t = s + NS * q
        buf = bufs[q % 2]
        sem = sems[q % 2]

        @pl.when(t < NCHUNK)
        def _(t=t, buf=buf, sem=sem, q=q):
            if q >= 2:
                pltpu.make_async_copy(
                    buf, part_hbm.at[c, pl.ds(0, C)], sem
                ).wait()
            pltpu.sync_copy(acc.at[pl.ds(t * C, C)], buf)
            pltpu.async_copy(buf, part_hbm.at[c, pl.ds(t * C, C)], sem)

    for q in range(max(NQ - 2, 0), NQ):
        t = s + NS * q

        @pl.when(t < NCHUNK)
        def _(buf=bufs[q % 2], sem=sems[q % 2]):
            pltpu.make_async_copy(buf, part_hbm.at[c, pl.ds(0, C)], sem).wait()


def _final_body(part_ref, innorm_ref, out_ref):
    agg = part_ref[0] + part_ref[1]
    out_ref[...] = agg * (innorm_ref[...] * ALPHA)[:, None]


def _final_call(parts, in_norm):
    return pl.pallas_call(
        _final_body,
        out_shape=jax.ShapeDtypeStruct((N, D), jnp.float32),
    )(parts, in_norm)


def kernel(emb, edge_index):
    src = edge_index[0]
    dst = edge_index[1]
    degs = _deg_kernel(src, dst)
    m, in_norm = _norm_call(emb, degs)
    src2d = src.reshape(E // C, C)
    dst2d = dst.reshape(E // C, C)
    parts = _scatter_kernel(m, src2d, dst2d)
    return _final_call(parts, in_norm)


# speedup vs baseline: 2.0022x; 1.0093x over previous
import functools

import jax
import jax.numpy as jnp
from jax import lax
from jax.experimental import pallas as pl
from jax.experimental.pallas import tpu as pltpu
from jax.experimental.pallas import tpu_sc as plsc

N = 10000
E = 320000
D = 128
N_LAYERS = 3
ALPHA = sum(1.0 / (1 + k) for k in range(N_LAYERS))

NC, NS, L = 2, 16, 16
NW = NC * NS
EPW = E // NW
C = 80
NCHUNK = EPW // C
RPT = N // NS
ZR = 125
NZ = RPT // ZR

_mesh = plsc.VectorSubcoreMesh(
    core_axis_name="c", subcore_axis_name="s", num_cores=NC, num_subcores=NS
)
_sc_params = pltpu.CompilerParams(
    needs_layout_passes=False, use_tc_tiling_on_sc=False
)


@functools.partial(
    pl.kernel,
    out_type=jax.ShapeDtypeStruct((NW, 2, N), jnp.float32),
    mesh=_mesh,
    scratch_types=[
        pltpu.VMEM((EPW,), jnp.int32),
        pltpu.VMEM((EPW,), jnp.int32),
        pltpu.VMEM((N,), jnp.float32),
        pltpu.VMEM((N,), jnp.float32),
        pltpu.SemaphoreType.DMA,
        pltpu.SemaphoreType.DMA,
    ],
    compiler_params=_sc_params,
)
def _deg_kernel(src_hbm, dst_hbm, deg_hbm, ev_s, ev_d, cnt_s, cnt_d, dma_s, dma_d):
    c = lax.axis_index("c")
    s = lax.axis_index("s")
    wid = c * NS + s
    base = wid * EPW
    zeros = jnp.zeros((L,), jnp.float32)
    ones = jnp.ones((L,), jnp.float32)

    pltpu.async_copy(src_hbm.at[pl.ds(base, EPW)], ev_s, dma_s)
    pltpu.async_copy(dst_hbm.at[pl.ds(base, EPW)], ev_d, dma_d)

    def zero_body(g, _):
        for u in range(5):
            cnt_s[pl.ds((5 * g + u) * L, L)] = zeros
            cnt_d[pl.ds((5 * g + u) * L, L)] = zeros
        return 0

    lax.fori_loop(0, N // L // 5, zero_body, 0)

    for cnt, eh, ev, dma in (
        (cnt_s, src_hbm, ev_s, dma_s),
        (cnt_d, dst_hbm, ev_d, dma_d),
    ):
        pltpu.make_async_copy(eh.at[pl.ds(base, EPW)], ev, dma).wait()

        def acc_body(g, _, cnt=cnt, ev=ev):
            for u in range(5):
                idx = ev[pl.ds((5 * g + u) * L, L)]
                plsc.addupdate_scatter(cnt, [idx], ones)
            return 0

        lax.fori_loop(0, EPW // L // 5, acc_body, 0)

    pltpu.async_copy(cnt_s, deg_hbm.at[wid, 0], dma_s)
    pltpu.async_copy(cnt_d, deg_hbm.at[wid, 1], dma_d)
    pltpu.make_async_copy(cnt_s, deg_hbm.at[wid, 0], dma_s).wait()
    pltpu.make_async_copy(cnt_d, deg_hbm.at[wid, 1], dma_d).wait()


def _norm_body(emb_ref, deg_ref, m_ref, innorm_ref):
    out_deg = jnp.sum(deg_ref[:, 0, :], axis=0)
    in_deg = jnp.sum(deg_ref[:, 1, :], axis=0)
    out_norm = lax.rsqrt(jnp.maximum(out_deg, 1.0))
    innorm_ref[...] = lax.rsqrt(jnp.maximum(in_deg, 1.0))
    m_ref[...] = emb_ref[...] * out_norm[:, None]


def _norm_call(emb, degs):
    return pl.pallas_call(
        _norm_body,
        out_shape=[
            jax.ShapeDtypeStruct((N, D), jnp.float32),
            jax.ShapeDtypeStruct((N,), jnp.float32),
        ],
    )(emb, degs)


@functools.partial(
    pl.kernel,
    out_type=jax.ShapeDtypeStruct((NC, N, D), jnp.float32),
    mesh=_mesh,
    scratch_types=[
        pltpu.VMEM_SHARED((N, D), jnp.float32),
        pltpu.VMEM((NCHUNK, C), jnp.int32),
        pltpu.VMEM((NCHUNK, C), jnp.int32),
        pltpu.VMEM((C, D), jnp.float32),
        pltpu.VMEM((C, D), jnp.float32),
        pltpu.VMEM((C, D), jnp.float32),
        pltpu.SemaphoreType.DMA,
        pltpu.SemaphoreType.DMA,
        pltpu.SemaphoreType.DMA,
    ],
    compiler_params=_sc_params,
)
def _scatter_kernel(
    m_hbm, src_hbm, dst_hbm, part_hbm,
    acc, src_all, dst_all, rows0, rows1, rows2, sem0, sem1, sem2,
):
    c = lax.axis_index("c")
    s = lax.axis_index("s")
    wid = c * NS + s
    zeros = jnp.zeros((L,), jnp.float32)

    pltpu.sync_copy(src_hbm.at[pl.ds(wid * NCHUNK, NCHUNK)], src_all)
    pltpu.sync_copy(dst_hbm.at[pl.ds(wid * NCHUNK, NCHUNK)], dst_all)

    def zb_body(i, _):
        def zb_inner(j, _):
            rows0[i, pl.ds(j * L, L)] = zeros
            return 0

        lax.fori_loop(0, D // L, zb_inner, 0)
        return 0

    lax.fori_loop(0, C, zb_body, 0)

    for q in range((NCHUNK + NS - 1) // NS):
        t = s + NS * q

        @pl.when(t < NCHUNK)
        def _():
            pltpu.async_copy(rows0, acc.at[pl.ds(t * C, C)], sem2)

    for q in range((NCHUNK + NS - 1) // NS):
        t = s + NS * q

        @pl.when(t < NCHUNK)
        def _():
            pltpu.make_async_copy(rows0, acc.at[pl.ds(0, C)], sem2).wait()

    plsc.subcore_barrier()

    bufs = (rows0, rows1, rows2)
    sems = (sem0, sem1, sem2)
    NB = 3

    def gstart(k, b):
        pltpu.async_copy(m_hbm.at[src_all.at[k]], bufs[b], sems[b])

    def gwait(b):
        pltpu.make_async_copy(m_hbm.at[src_all.at[0]], bufs[b], sems[b]).wait()

    def scat(k, b):
        pltpu.sync_copy(bufs[b], acc.at[dst_all.at[k]], add=True)

    gstart(0, 0)
    gstart(1, 1)

    def ring_body(q, _):
        for r in range(NB):
            k = NB * q + r
            gwait(r)

            @pl.when(k + NB - 1 < NCHUNK)
            def _():
                gstart(k + NB - 1, (r + NB - 1) % NB)

            scat(k, r)
        return 0

    lax.fori_loop(0, NCHUNK // NB, ring_body, 0)
    for k in range((NCHUNK // NB) * NB, NCHUNK):
        gwait(k % NB)
        scat(k, k % NB)
    plsc.subcore_barrier()

    NQ = (NCHUNK + NS - 1) // NS
    for q in range(NQ):
        t = s + NS * q
        buf = bufs[q % 2]
        sem = sems[q % 2]

        @pl.when(t < NCHUNK)
        def _(t=t, buf=buf, sem=sem, q=q):
            if q >= 2:
                pltpu.make_async_copy(
                    buf, part_hbm.at[c, pl.ds(0, C)], sem
                ).wait()
            pltpu.sync_copy(acc.at[pl.ds(t * C, C)], buf)
            pltpu.async_copy(buf, part_hbm.at[c, pl.ds(t * C, C)], sem)

    for q in range(max(NQ - 2, 0), NQ):
        t = s + NS * q

        @pl.when(t < NCHUNK)
        def _(buf=bufs[q % 2], sem=sems[q % 2]):
            pltpu.make_async_copy(buf, part_hbm.at[c, pl.ds(0, C)], sem).wait()


def _final_body(part_ref, innorm_ref, out_ref):
    agg = part_ref[0] + part_ref[1]
    out_ref[...] = agg * (innorm_ref[...] * ALPHA)[:, None]


def _final_call(parts, in_norm):
    return pl.pallas_call(
        _final_body,
        out_shape=jax.ShapeDtypeStruct((N, D), jnp.float32),
    )(parts, in_norm)


def kernel(emb, edge_index):
    src = edge_index[0]
    dst = edge_index[1]
    degs = _deg_kernel(src, dst)
    m, in_norm = _norm_call(emb, degs)
    src2d = src.reshape(E // C, C)
    dst2d = dst.reshape(E // C, C)
    parts = _scatter_kernel(m, src2d, dst2d)
    return _final_call(parts, in_norm)
